# baseline (speedup 1.0000x reference)
import jax
import jax.numpy as jnp
from jax import lax
from jax.experimental import pallas as pl
from jax.experimental.pallas import tpu as pltpu
from jax.experimental.pallas import tpu_sc as plsc

N_USERS = 1000000
N_ITEMS = 1000000
N_FACTORS = 64
BATCH = 16384
Y_SCALE = 5.0

NC = 2
NS = 16
NW = NC * NS
B_PER_W = BATCH // NW
IDX_CHUNK = 128
N_CHUNKS = B_PER_W // IDX_CHUNK
L = 16
WROWS = 256
N_WAVES = B_PER_W // WROWS
STRIDE = WROWS + 1
PAIR_ROWS = N_USERS // 2


def _dot_body(users_hbm, items_hbm, uw2_hbm, iw2_hbm, dots_hbm,
              uidx_v, iidx_v, upair_v, ipair_v, urows_v, irows_v,
              stage_v, dots_v, sem):
    wid = lax.axis_index("s") * NC + lax.axis_index("c")
    base = wid * B_PER_W

    for c in range(N_CHUNKS):
        pltpu.sync_copy(users_hbm.at[pl.ds(base + c * IDX_CHUNK, IDX_CHUNK)],
                        uidx_v.at[c])
        pltpu.sync_copy(items_hbm.at[pl.ds(base + c * IDX_CHUNK, IDX_CHUNK)],
                        iidx_v.at[c])

    def shift_body(g, carry):
        c, o = g // (IDX_CHUNK // L), (g % (IDX_CHUNK // L)) * L
        upair_v[c, pl.ds(o, L)] = lax.shift_right_logical(
            uidx_v[c, pl.ds(o, L)], 1)
        ipair_v[c, pl.ds(o, L)] = lax.shift_right_logical(
            iidx_v[c, pl.ds(o, L)], 1)
        return carry

    lax.fori_loop(0, B_PER_W // L, shift_body, 0)

    lane = lax.iota(jnp.int32, L)
    lane_base = lane * STRIDE

    for w in range(N_WAVES):
        copies = []
        for k in range(WROWS // IDX_CHUNK):
            c = w * (WROWS // IDX_CHUNK) + k
            sl = pl.ds(k * IDX_CHUNK, IDX_CHUNK)
            copies.append(pltpu.async_copy(uw2_hbm.at[upair_v.at[c]],
                                           urows_v.at[sl], sem))
            copies.append(pltpu.async_copy(iw2_hbm.at[ipair_v.at[c]],
                                           irows_v.at[sl], sem))
        for cp in copies:
            cp.wait()

        def row_body(g, carry):
            b0 = w * WROWS + g * L
            c, o = b0 // IDX_CHUNK, b0 % IDX_CHUNK
            upar = lax.bitwise_and(uidx_v[c, pl.ds(o, L)], 1) * N_FACTORS
            ipar = lax.bitwise_and(iidx_v[c, pl.ds(o, L)], 1) * N_FACTORS
            for j in range(L):
                r = g * L + j
                uo = upar[j]
                io = ipar[j]
                p = (urows_v[r, pl.ds(uo, L)] * irows_v[r, pl.ds(io, L)]
                     + urows_v[r, pl.ds(uo + L, L)]
                     * irows_v[r, pl.ds(io + L, L)]
                     + urows_v[r, pl.ds(uo + 2 * L, L)]
                     * irows_v[r, pl.ds(io + 2 * L, L)]
                     + urows_v[r, pl.ds(uo + 3 * L, L)]
                     * irows_v[r, pl.ds(io + 3 * L, L)])
                plsc.store_scatter(stage_v, [lane_base + r], p)
            return carry

        lax.fori_loop(0, WROWS // L, row_body, 0)

        def sum_body(g, carry):
            acc = stage_v[pl.ds(g * L, L)]
            for l in range(1, L):
                acc = acc + stage_v[pl.ds(l * STRIDE + g * L, L)]
            dots_v[pl.ds(w * WROWS + g * L, L)] = acc
            return carry

        lax.fori_loop(0, WROWS // L, sum_body, 0)

    pltpu.sync_copy(dots_v, dots_hbm.at[pl.ds(base, B_PER_W)])


def _bias_body(users_hbm, items_hbm, ub_hbm, ib_hbm, dots_hbm, out_hbm,
               uidx_v, iidx_v, ub_v, ib_v, dots_v, out_v, sem):
    wid = lax.axis_index("s") * NC + lax.axis_index("c")
    base = wid * B_PER_W

    for c in range(N_CHUNKS):
        pltpu.sync_copy(users_hbm.at[pl.ds(base + c * IDX_CHUNK, IDX_CHUNK)],
                        uidx_v.at[c])
        pltpu.sync_copy(items_hbm.at[pl.ds(base + c * IDX_CHUNK, IDX_CHUNK)],
                        iidx_v.at[c])
    pltpu.sync_copy(dots_hbm.at[pl.ds(base, B_PER_W)], dots_v)

    copies = []
    for c in range(N_CHUNKS):
        sl = pl.ds(c * IDX_CHUNK, IDX_CHUNK)
        copies.append(pltpu.async_copy(ub_hbm.at[uidx_v.at[c]],
                                       ub_v.at[sl], sem))
        copies.append(pltpu.async_copy(ib_hbm.at[iidx_v.at[c]],
                                       ib_v.at[sl], sem))
    for cp in copies:
        cp.wait()

    def epi_body(g, carry):
        s = pl.ds(g * L, L)
        x = dots_v[s] + ub_v[s] + ib_v[s]
        out_v[s] = Y_SCALE / (1.0 + jnp.exp(-x))
        return carry

    lax.fori_loop(0, B_PER_W // L, epi_body, 0)

    pltpu.sync_copy(out_v, out_hbm.at[pl.ds(base, B_PER_W)])


@jax.jit
def _run(users, items, u_weight, i_weight, u_bias, i_bias):
    mesh = plsc.VectorSubcoreMesh(core_axis_name="c", subcore_axis_name="s",
                                  num_cores=NC, num_subcores=NS)
    dot_f = pl.kernel(
        _dot_body,
        out_type=jax.ShapeDtypeStruct((BATCH,), jnp.float32),
        mesh=mesh,
        compiler_params=pltpu.CompilerParams(needs_layout_passes=False,
                                             use_tc_tiling_on_sc=True),
        scratch_types=[
            pltpu.VMEM((N_CHUNKS, IDX_CHUNK), jnp.int32),
            pltpu.VMEM((N_CHUNKS, IDX_CHUNK), jnp.int32),
            pltpu.VMEM((N_CHUNKS, IDX_CHUNK), jnp.int32),
            pltpu.VMEM((N_CHUNKS, IDX_CHUNK), jnp.int32),
            pltpu.VMEM((WROWS, 2 * N_FACTORS), jnp.float32),
            pltpu.VMEM((WROWS, 2 * N_FACTORS), jnp.float32),
            pltpu.VMEM((L * STRIDE,), jnp.float32),
            pltpu.VMEM((B_PER_W,), jnp.float32),
            pltpu.SemaphoreType.DMA,
        ],
    )
    dots = dot_f(users, items,
                 u_weight.reshape(PAIR_ROWS, 2 * N_FACTORS),
                 i_weight.reshape(PAIR_ROWS, 2 * N_FACTORS))

    bias_f = pl.kernel(
        _bias_body,
        out_type=jax.ShapeDtypeStruct((BATCH,), jnp.float32),
        mesh=mesh,
        compiler_params=pltpu.CompilerParams(needs_layout_passes=False,
                                             use_tc_tiling_on_sc=False),
        scratch_types=[
            pltpu.VMEM((N_CHUNKS, IDX_CHUNK), jnp.int32),
            pltpu.VMEM((N_CHUNKS, IDX_CHUNK), jnp.int32),
            pltpu.VMEM((B_PER_W,), jnp.float32),
            pltpu.VMEM((B_PER_W,), jnp.float32),
            pltpu.VMEM((B_PER_W,), jnp.float32),
            pltpu.VMEM((B_PER_W,), jnp.float32),
            pltpu.SemaphoreType.DMA,
        ],
    )
    return bias_f(users, items, u_bias.reshape(N_USERS),
                  i_bias.reshape(N_ITEMS), dots)


def kernel(users, items, u_weight, i_weight, u_bias, i_bias):
    return _run(users.astype(jnp.int32), items.astype(jnp.int32),
                u_weight, i_weight, u_bias, i_bias)
